# baseline (speedup 1.0000x reference)
import functools

import jax
import jax.numpy as jnp
from jax import lax
from jax.experimental import pallas as pl
from jax.experimental.pallas import tpu as pltpu
from jax.experimental.pallas import tpu_sc as plsc

_RADIUS = 0.2
_K = 32
_B, _N, _S, _C = 4, 8192, 1024, 64
_NCH = 3 + 8 + _C

_NC, _NS, _L = 2, 16, 16
_NW = _NC * _NS
_SPW = (_B * _S) // _NW
_WPB = _NW // _B
_G = 16
_U = 32


def _rsqrt_nr(x):
    i = plsc.bitcast(x, jnp.int32)
    i = jnp.int32(0x5F3759DF) - lax.shift_right_logical(
        i, jnp.full((_L,), 1, jnp.int32))
    y = plsc.bitcast(i, jnp.float32)
    half, three_half = jnp.float32(0.5), jnp.float32(1.5)
    for _ in range(3):
        y = y * (three_half - half * x * y * y)
    return y


def _splat_at(ref, pos):
    return plsc.load_gather(ref, [jnp.full((_L,), pos, jnp.int32)])


def _sc_body(xyzT, newT, featT, out, xyz_v, new_v, idxbuf, idx2, rows2,
             tiles, sem_g0, sem_g1, sem_out):
    cid = lax.axis_index("c")
    sid = lax.axis_index("s")
    wid = sid * _NC + cid
    b = wid // _WPB
    s_base = (wid % _WPB) * _SPW

    pltpu.sync_copy(xyzT.at[b, 0], xyz_v)
    for comp in range(3):
        pltpu.sync_copy(newT.at[b * 3 + comp, 0, pl.ds(s_base, _SPW)],
                        new_v.at[pl.ds(comp * _SPW, _SPW)])

    r2 = jnp.float32(_RADIUS * _RADIUS)
    inv_r = jnp.float32(1.0 / _RADIUS)
    inv_k = jnp.float32(1.0 / _K)
    iota = lax.iota(jnp.int32, _L)
    zeros_i = jnp.zeros((_L,), jnp.int32)
    ones_i = jnp.ones((_L,), jnp.int32)
    feat_base = b * _N

    def _flush_copy(tp, tf):
        return pltpu.make_async_copy(
            tiles.at[tp],
            out.at[b, :, pl.ds((s_base + tf * _G) * _K, _G * _K)],
            sem_out)

    def _gather_copy(slot):
        sem = sem_g0 if slot == 0 else sem_g1
        return pltpu.make_async_copy(featT.at[idx2.at[slot]],
                                     rows2.at[slot], sem)

    rowB = iota + _L

    def _drain(p):
        pg = p // _G
        pgi = p - pg * _G
        ptp = lax.rem(pg, 2)
        pslot = lax.rem(p, 2)
        pobase = pgi * _K
        lax.cond(pslot == 0,
                 lambda: _gather_copy(0).wait(),
                 lambda: _gather_copy(1).wait())
        psl_v = jnp.full((_L,), pslot, jnp.int32)
        for c in range(_C):
            colv = jnp.full((_L,), c, jnp.int32)
            tiles[ptp, 11 + c, pl.ds(pobase, _L)] = \
                plsc.load_gather(rows2, [psl_v, iota, colv])
            tiles[ptp, 11 + c, pl.ds(pobase + _L, _L)] = \
                plsc.load_gather(rows2, [psl_v, rowB, colv])
        return pg, ptp

    def centroid_body(s_local, carry_g):
        g = s_local // _G
        gi = s_local - g * _G
        tp = lax.rem(g, 2)
        slot = lax.rem(s_local, 2)
        out_tile = tiles.at[tp]

        @pl.when(jnp.logical_and(gi == 0, g >= 2))
        def _():
            _flush_copy(tp, g - 2).wait()

        cx = _splat_at(new_v, s_local)
        cy = _splat_at(new_v, _SPW + s_local)
        cz = _splat_at(new_v, 2 * _SPW + s_local)
        idxbuf[pl.ds(0, _L)] = zeros_i

        def cond_fn(c):
            n0, cnt_v = c
            return jnp.logical_and(cnt_v[0] < _K, n0 < _N)

        def body_fn(c):
            n0, cnt_v = c
            masks = []
            for w in range(_U):
                off = n0 + w * _L
                xv = xyz_v[pl.ds(off, _L)]
                yv = xyz_v[pl.ds(_N + off, _L)]
                zv = xyz_v[pl.ds(2 * _N + off, _L)]
                dx = xv - cx
                dy = yv - cy
                dz = zv - cz
                d2 = dx * dx + dy * dy + dz * dz
                masks.append((d2 < r2, iota + off))
            pcs = [plsc.all_reduce_population_count(m) for m, _ in masks]
            css = [plsc.cumsum(ones_i, mask=m) for m, _ in masks]
            base = cnt_v
            for w in range(_U):
                m, vals = masks[w]
                plsc.store_scatter(idxbuf, [css[w] + base - 1], vals,
                                   mask=m)
                base = base + pcs[w]
            return n0 + _U * _L, base

        _, cnt_v = lax.while_loop(cond_fn, body_fn,
                                  (jnp.int32(0), zeros_i))

        @pl.when(s_local >= 2)
        def _():
            pg, ptp = _drain(s_local - 2)

            @pl.when(gi == 1)
            def _():
                _flush_copy(ptp, pg).start()

        first_v = plsc.load_gather(idxbuf, [zeros_i])
        sels = []
        for q in range(_K // _L):
            kvec = iota + q * _L
            vals = idxbuf[pl.ds(q * _L, _L)]
            sel = jnp.where(kvec < cnt_v, vals, first_v)
            idx2[slot, pl.ds(q * _L, _L)] = sel + feat_base
            sels.append(sel)
        lax.cond(slot == 0,
                 lambda: _gather_copy(0).start(),
                 lambda: _gather_copy(1).start())

        obase = gi * _K
        sx = sy = sz = sxx = syy = szz = sd = maxd = None
        for q in range(_K // _L):
            sel = sels[q]
            px = plsc.load_gather(xyz_v, [sel])
            py = plsc.load_gather(xyz_v, [sel + _N])
            pz = plsc.load_gather(xyz_v, [sel + 2 * _N])
            rx = px - cx
            ry = py - cy
            rz = pz - cz
            out_tile[0, pl.ds(obase + q * _L, _L)] = rx
            out_tile[1, pl.ds(obase + q * _L, _L)] = ry
            out_tile[2, pl.ds(obase + q * _L, _L)] = rz
            nx = rx * inv_r
            ny = ry * inv_r
            nz = rz * inv_r
            x2, y2, z2 = nx * nx, ny * ny, nz * nz
            dn2 = x2 + y2 + z2 + jnp.float32(1e-12)
            dist = dn2 * _rsqrt_nr(dn2)
            if q == 0:
                sx, sy, sz, sxx, syy, szz = nx, ny, nz, x2, y2, z2
                sd, maxd = dist, dist
            else:
                sx, sy, sz = sx + nx, sy + ny, sz + nz
                sxx, syy, szz = sxx + x2, syy + y2, szz + z2
                sd = sd + dist
                maxd = jnp.maximum(maxd, dist)

        mean_rows = []
        std_rows = []
        for sv, sq in ((sx, sxx), (sy, syy), (sz, szz)):
            mv = jnp.full((_L,), jnp.sum(sv), jnp.float32) * inv_k
            qv = jnp.full((_L,), jnp.sum(sq), jnp.float32) * inv_k
            var = jnp.maximum(qv - mv * mv, jnp.float32(0.0)) \
                + jnp.float32(1e-20)
            mean_rows.append(mv)
            std_rows.append(var * _rsqrt_nr(var))
        md_v = jnp.full((_L,), jnp.sum(sd), jnp.float32) * inv_k
        mx_v = jnp.full((_L,), jnp.max(maxd), jnp.float32)
        for ci, row in enumerate(mean_rows + std_rows + [md_v, mx_v]):
            out_tile[3 + ci, pl.ds(obase, _L)] = row
            out_tile[3 + ci, pl.ds(obase + _L, _L)] = row

        return carry_g

    lax.fori_loop(0, _SPW, centroid_body, 0)
    ngroups = _SPW // _G
    _drain(_SPW - 2)
    _drain(_SPW - 1)
    _flush_copy((ngroups - 1) % 2, ngroups - 1).start()
    _flush_copy((ngroups - 2) % 2, ngroups - 2).wait()
    _flush_copy((ngroups - 1) % 2, ngroups - 1).wait()


def kernel(xyz, new_xyz, features):
    xyzT = jnp.transpose(xyz, (0, 2, 1)).reshape(_B, 1, 3 * _N)
    newT = jnp.transpose(new_xyz, (0, 2, 1)).reshape(_B * 3, 1, _S)
    featT = jnp.transpose(features, (0, 2, 1)).reshape(_B * _N, _C)
    mesh = plsc.VectorSubcoreMesh(core_axis_name="c", subcore_axis_name="s")
    f = functools.partial(
        pl.kernel,
        out_type=jax.ShapeDtypeStruct((_B, _NCH, _S * _K), jnp.float32),
        mesh=mesh,
        compiler_params=pltpu.CompilerParams(needs_layout_passes=False, use_tc_tiling_on_sc=False),
        scratch_types=[
            pltpu.VMEM((3 * _N,), jnp.float32),
            pltpu.VMEM((3 * _SPW,), jnp.float32),
            pltpu.VMEM((544,), jnp.int32),
            pltpu.VMEM((2, _K), jnp.int32),
            pltpu.VMEM((2, _K, _C), jnp.float32),
            pltpu.VMEM((2, _NCH, _G * _K), jnp.float32),
            pltpu.SemaphoreType.DMA,
            pltpu.SemaphoreType.DMA,
            pltpu.SemaphoreType.DMA,
        ],
    )(_sc_body)
    return f(xyzT, newT, featT).reshape(_B, _NCH, _S, _K)


# speedup vs baseline: 1.2900x; 1.2900x over previous
import functools

import jax
import jax.numpy as jnp
from jax import lax
from jax.experimental import pallas as pl
from jax.experimental.pallas import tpu as pltpu
from jax.experimental.pallas import tpu_sc as plsc

_RADIUS = 0.2
_K = 32
_B, _N, _S, _C = 4, 8192, 1024, 64
_NCH = 3 + 8 + _C

_CP = _C + 8
_NC, _NS, _L = 2, 16, 16
_NW = _NC * _NS
_SPW = (_B * _S) // _NW
_WPB = _NW // _B
_G = 16
_U = 16


def _rsqrt_nr(x):
    i = plsc.bitcast(x, jnp.int32)
    i = jnp.int32(0x5F3759DF) - lax.shift_right_logical(
        i, jnp.full((_L,), 1, jnp.int32))
    y = plsc.bitcast(i, jnp.float32)
    half, three_half = jnp.float32(0.5), jnp.float32(1.5)
    for _ in range(3):
        y = y * (three_half - half * x * y * y)
    return y


def _splat_at(ref, pos):
    return plsc.load_gather(ref, [jnp.full((_L,), pos, jnp.int32)])


def _sc_body(xyzT, newT, featT, out, xyz_v, new_v, idxbuf, idx2, rows2,
             tiles, sem_g0, sem_g1, sem_out):
    cid = lax.axis_index("c")
    sid = lax.axis_index("s")
    wid = sid * _NC + cid
    b = wid // _WPB
    s_base = (wid % _WPB) * _SPW

    pltpu.sync_copy(xyzT.at[b, 0], xyz_v)
    for comp in range(3):
        pltpu.sync_copy(newT.at[b * 3 + comp, 0, pl.ds(s_base, _SPW)],
                        new_v.at[pl.ds(comp * _SPW, _SPW)])

    r2 = jnp.float32(_RADIUS * _RADIUS)
    inv_r = jnp.float32(1.0 / _RADIUS)
    inv_k = jnp.float32(1.0 / _K)
    iota = lax.iota(jnp.int32, _L)
    zeros_i = jnp.zeros((_L,), jnp.int32)
    ones_i = jnp.ones((_L,), jnp.int32)
    feat_base = b * _N

    def _flush_copy(tp, tf):
        return pltpu.make_async_copy(
            tiles.at[tp],
            out.at[b, :, pl.ds((s_base + tf * _G) * _K, _G * _K)],
            sem_out)

    def _gather_copy(slot):
        sem = sem_g0 if slot == 0 else sem_g1
        return pltpu.make_async_copy(featT.at[idx2.at[slot]],
                                     rows2.at[slot], sem)

    rowB = iota + _L

    def _drain(p):
        pg = p // _G
        pgi = p - pg * _G
        ptp = lax.rem(pg, 2)
        pslot = lax.rem(p, 2)
        pobase = pgi * _K
        lax.cond(pslot == 0,
                 lambda: _gather_copy(0).wait(),
                 lambda: _gather_copy(1).wait())
        psl_v = jnp.full((_L,), pslot, jnp.int32)
        for c in range(_C):
            colv = jnp.full((_L,), c, jnp.int32)
            tiles[ptp, 11 + c, pl.ds(pobase, _L)] = \
                plsc.load_gather(rows2, [psl_v, iota, colv])
            tiles[ptp, 11 + c, pl.ds(pobase + _L, _L)] = \
                plsc.load_gather(rows2, [psl_v, rowB, colv])
        return pg, ptp

    def centroid_body(s_local, carry_g):
        g = s_local // _G
        gi = s_local - g * _G
        tp = lax.rem(g, 2)
        slot = lax.rem(s_local, 2)
        out_tile = tiles.at[tp]

        @pl.when(jnp.logical_and(gi == 0, g >= 2))
        def _():
            _flush_copy(tp, g - 2).wait()

        cx = _splat_at(new_v, s_local)
        cy = _splat_at(new_v, _SPW + s_local)
        cz = _splat_at(new_v, 2 * _SPW + s_local)
        idxbuf[pl.ds(0, _L)] = zeros_i

        def cond_fn(c):
            n0, cnt_v = c
            return jnp.logical_and(cnt_v[0] < _K, n0 < _N)

        def body_fn(c):
            n0, cnt_v = c
            masks = []
            for w in range(_U):
                off = n0 + w * _L
                xv = xyz_v[pl.ds(off, _L)]
                yv = xyz_v[pl.ds(_N + off, _L)]
                zv = xyz_v[pl.ds(2 * _N + off, _L)]
                dx = xv - cx
                dy = yv - cy
                dz = zv - cz
                d2 = dx * dx + dy * dy + dz * dz
                masks.append((d2 < r2, iota + off))
            pcs = [plsc.all_reduce_population_count(m) for m, _ in masks]
            css = [plsc.cumsum(ones_i, mask=m) for m, _ in masks]
            base = cnt_v
            for w in range(_U):
                m, vals = masks[w]
                plsc.store_scatter(idxbuf, [css[w] + base - 1], vals,
                                   mask=m)
                base = base + pcs[w]
            return n0 + _U * _L, base

        _, cnt_v = lax.while_loop(cond_fn, body_fn,
                                  (jnp.int32(0), zeros_i))

        @pl.when(s_local >= 2)
        def _():
            pg, ptp = _drain(s_local - 2)

            @pl.when(gi == 1)
            def _():
                _flush_copy(ptp, pg).start()

        first_v = plsc.load_gather(idxbuf, [zeros_i])
        sels = []
        for q in range(_K // _L):
            kvec = iota + q * _L
            vals = idxbuf[pl.ds(q * _L, _L)]
            sel = jnp.where(kvec < cnt_v, vals, first_v)
            idx2[slot, pl.ds(q * _L, _L)] = sel + feat_base
            sels.append(sel)
        lax.cond(slot == 0,
                 lambda: _gather_copy(0).start(),
                 lambda: _gather_copy(1).start())

        obase = gi * _K
        sx = sy = sz = sxx = syy = szz = sd = maxd = None
        for q in range(_K // _L):
            sel = sels[q]
            px = plsc.load_gather(xyz_v, [sel])
            py = plsc.load_gather(xyz_v, [sel + _N])
            pz = plsc.load_gather(xyz_v, [sel + 2 * _N])
            rx = px - cx
            ry = py - cy
            rz = pz - cz
            out_tile[0, pl.ds(obase + q * _L, _L)] = rx
            out_tile[1, pl.ds(obase + q * _L, _L)] = ry
            out_tile[2, pl.ds(obase + q * _L, _L)] = rz
            nx = rx * inv_r
            ny = ry * inv_r
            nz = rz * inv_r
            x2, y2, z2 = nx * nx, ny * ny, nz * nz
            dn2 = x2 + y2 + z2 + jnp.float32(1e-12)
            dist = dn2 * _rsqrt_nr(dn2)
            if q == 0:
                sx, sy, sz, sxx, syy, szz = nx, ny, nz, x2, y2, z2
                sd, maxd = dist, dist
            else:
                sx, sy, sz = sx + nx, sy + ny, sz + nz
                sxx, syy, szz = sxx + x2, syy + y2, szz + z2
                sd = sd + dist
                maxd = jnp.maximum(maxd, dist)

        mean_rows = []
        std_rows = []
        for sv, sq in ((sx, sxx), (sy, syy), (sz, szz)):
            mv = jnp.full((_L,), jnp.sum(sv), jnp.float32) * inv_k
            qv = jnp.full((_L,), jnp.sum(sq), jnp.float32) * inv_k
            var = jnp.maximum(qv - mv * mv, jnp.float32(0.0)) \
                + jnp.float32(1e-20)
            mean_rows.append(mv)
            std_rows.append(var * _rsqrt_nr(var))
        md_v = jnp.full((_L,), jnp.sum(sd), jnp.float32) * inv_k
        mx_v = jnp.full((_L,), jnp.max(maxd), jnp.float32)
        for ci, row in enumerate(mean_rows + std_rows + [md_v, mx_v]):
            out_tile[3 + ci, pl.ds(obase, _L)] = row
            out_tile[3 + ci, pl.ds(obase + _L, _L)] = row

        return carry_g

    lax.fori_loop(0, _SPW, centroid_body, 0)
    ngroups = _SPW // _G
    _drain(_SPW - 2)
    _drain(_SPW - 1)
    _flush_copy((ngroups - 1) % 2, ngroups - 1).start()
    _flush_copy((ngroups - 2) % 2, ngroups - 2).wait()
    _flush_copy((ngroups - 1) % 2, ngroups - 1).wait()


def kernel(xyz, new_xyz, features):
    xyzT = jnp.transpose(xyz, (0, 2, 1)).reshape(_B, 1, 3 * _N)
    newT = jnp.transpose(new_xyz, (0, 2, 1)).reshape(_B * 3, 1, _S)
    featT = jnp.transpose(features, (0, 2, 1)).reshape(_B * _N, _C)
    featT = jnp.pad(featT, ((0, 0), (0, _CP - _C)))
    mesh = plsc.VectorSubcoreMesh(core_axis_name="c", subcore_axis_name="s")
    f = functools.partial(
        pl.kernel,
        out_type=jax.ShapeDtypeStruct((_B, _NCH, _S * _K), jnp.float32),
        mesh=mesh,
        compiler_params=pltpu.CompilerParams(needs_layout_passes=False, use_tc_tiling_on_sc=False),
        scratch_types=[
            pltpu.VMEM((3 * _N,), jnp.float32),
            pltpu.VMEM((3 * _SPW,), jnp.float32),
            pltpu.VMEM((320,), jnp.int32),
            pltpu.VMEM((2, _K), jnp.int32),
            pltpu.VMEM((2, _K, _CP), jnp.float32),
            pltpu.VMEM((2, _NCH, _G * _K), jnp.float32),
            pltpu.SemaphoreType.DMA,
            pltpu.SemaphoreType.DMA,
            pltpu.SemaphoreType.DMA,
        ],
    )(_sc_body)
    return f(xyzT, newT, featT).reshape(_B, _NCH, _S, _K)


# speedup vs baseline: 1.3013x; 1.0088x over previous
import functools

import jax
import jax.numpy as jnp
from jax import lax
from jax.experimental import pallas as pl
from jax.experimental.pallas import tpu as pltpu
from jax.experimental.pallas import tpu_sc as plsc

_RADIUS = 0.2
_K = 32
_B, _N, _S, _C = 4, 8192, 1024, 64
_NCH = 3 + 8 + _C

_CP = _C + 8
_NC, _NS, _L = 2, 16, 16
_NW = _NC * _NS
_SPW = (_B * _S) // _NW
_WPB = _NW // _B
_G = 16
_U = 16


def _rsqrt_nr(x):
    i = plsc.bitcast(x, jnp.int32)
    i = jnp.int32(0x5F3759DF) - lax.shift_right_logical(
        i, jnp.full((_L,), 1, jnp.int32))
    y = plsc.bitcast(i, jnp.float32)
    half, three_half = jnp.float32(0.5), jnp.float32(1.5)
    for _ in range(3):
        y = y * (three_half - half * x * y * y)
    return y


def _splat_at(ref, pos):
    return plsc.load_gather(ref, [jnp.full((_L,), pos, jnp.int32)])


def _sc_body(xyzT, newT, featT, out, xyz_v, new_v, idxbuf, idx2, rows2,
             tiles, sem_g0, sem_g1, sem_out):
    cid = lax.axis_index("c")
    sid = lax.axis_index("s")
    wid = sid * _NC + cid
    b = wid // _WPB
    s_base = (wid % _WPB) * _SPW

    pltpu.sync_copy(xyzT.at[b, 0], xyz_v)
    for comp in range(3):
        pltpu.sync_copy(newT.at[b * 3 + comp, 0, pl.ds(s_base, _SPW)],
                        new_v.at[pl.ds(comp * _SPW, _SPW)])

    r2 = jnp.float32(_RADIUS * _RADIUS)
    inv_r = jnp.float32(1.0 / _RADIUS)
    inv_k = jnp.float32(1.0 / _K)
    iota = lax.iota(jnp.int32, _L)
    zeros_i = jnp.zeros((_L,), jnp.int32)
    ones_i = jnp.ones((_L,), jnp.int32)
    feat_base = b * _N

    def _flush_copy(tp, tf):
        return pltpu.make_async_copy(
            tiles.at[tp],
            out.at[b, :, pl.ds((s_base + tf * _G) * _K, _G * _K)],
            sem_out)

    def _gather_copy(slot):
        sem = sem_g0 if slot == 0 else sem_g1
        return pltpu.make_async_copy(featT.at[idx2.at[slot]],
                                     rows2.at[slot], sem)

    rowB = iota + _L

    def _drain(p):
        pg = p // _G
        pgi = p - pg * _G
        ptp = lax.rem(pg, 2)
        pslot = lax.rem(p, 2)
        pobase = pgi * _K
        lax.cond(pslot == 0,
                 lambda: _gather_copy(0).wait(),
                 lambda: _gather_copy(1).wait())
        psl_v = jnp.full((_L,), pslot, jnp.int32)
        for c in range(_C):
            colv = jnp.full((_L,), c, jnp.int32)
            tiles[ptp, 11 + c, pl.ds(pobase, _L)] = \
                plsc.load_gather(rows2, [psl_v, iota, colv])
            tiles[ptp, 11 + c, pl.ds(pobase + _L, _L)] = \
                plsc.load_gather(rows2, [psl_v, rowB, colv])
        return pg, ptp

    def centroid_body(s_local, carry_g):
        g = s_local // _G
        gi = s_local - g * _G
        tp = lax.rem(g, 2)
        slot = lax.rem(s_local, 2)
        out_tile = tiles.at[tp]

        @pl.when(jnp.logical_and(gi == 0, g >= 2))
        def _():
            _flush_copy(tp, g - 2).wait()

        cx = _splat_at(new_v, s_local)
        cy = _splat_at(new_v, _SPW + s_local)
        cz = _splat_at(new_v, 2 * _SPW + s_local)
        idxbuf[pl.ds(0, _L)] = zeros_i

        def cond_fn(c):
            n0, cnt_v = c
            return jnp.logical_and(cnt_v[0] < _K, n0 < _N)

        def body_fn(c):
            n0, cnt_v = c
            masks = []
            for w in range(_U):
                off = n0 + w * _L
                xv = xyz_v[pl.ds(off, _L)]
                yv = xyz_v[pl.ds(_N + off, _L)]
                zv = xyz_v[pl.ds(2 * _N + off, _L)]
                dx = xv - cx
                dy = yv - cy
                dz = zv - cz
                d2 = dx * dx + dy * dy + dz * dz
                masks.append((d2 < r2, iota + off))
            pcs = [plsc.all_reduce_population_count(m) for m, _ in masks]
            css = [plsc.cumsum(ones_i, mask=m) for m, _ in masks]
            base = cnt_v
            for w in range(_U):
                m, vals = masks[w]
                plsc.store_scatter(idxbuf, [css[w] + base - 1], vals,
                                   mask=m)
                base = base + pcs[w]
            return n0 + _U * _L, base

        _, cnt_v = lax.while_loop(cond_fn, body_fn,
                                  (jnp.int32(0), zeros_i))

        @pl.when(s_local >= 2)
        def _():
            pg, ptp = _drain(s_local - 2)

            @pl.when(gi == 1)
            def _():
                _flush_copy(ptp, pg).start()

        first_v = plsc.load_gather(idxbuf, [zeros_i])
        sels = []
        for q in range(_K // _L):
            kvec = iota + q * _L
            vals = idxbuf[pl.ds(q * _L, _L)]
            sel = jnp.where(kvec < cnt_v, vals, first_v)
            idx2[slot, pl.ds(q * _L, _L)] = sel + feat_base
            sels.append(sel)
        lax.cond(slot == 0,
                 lambda: _gather_copy(0).start(),
                 lambda: _gather_copy(1).start())

        obase = gi * _K
        sx = sy = sz = sxx = syy = szz = sd = maxd = None
        for q in range(_K // _L):
            sel = sels[q]
            px = plsc.load_gather(xyz_v, [sel])
            py = plsc.load_gather(xyz_v, [sel + _N])
            pz = plsc.load_gather(xyz_v, [sel + 2 * _N])
            rx = px - cx
            ry = py - cy
            rz = pz - cz
            out_tile[0, pl.ds(obase + q * _L, _L)] = rx
            out_tile[1, pl.ds(obase + q * _L, _L)] = ry
            out_tile[2, pl.ds(obase + q * _L, _L)] = rz
            nx = rx * inv_r
            ny = ry * inv_r
            nz = rz * inv_r
            x2, y2, z2 = nx * nx, ny * ny, nz * nz
            dn2 = x2 + y2 + z2 + jnp.float32(1e-12)
            dist = dn2 * _rsqrt_nr(dn2)
            if q == 0:
                sx, sy, sz, sxx, syy, szz = nx, ny, nz, x2, y2, z2
                sd, maxd = dist, dist
            else:
                sx, sy, sz = sx + nx, sy + ny, sz + nz
                sxx, syy, szz = sxx + x2, syy + y2, szz + z2
                sd = sd + dist
                maxd = jnp.maximum(maxd, dist)

        mean_rows = []
        std_rows = []
        for sv, sq in ((sx, sxx), (sy, syy), (sz, szz)):
            mv = jnp.full((_L,), jnp.sum(sv), jnp.float32) * inv_k
            qv = jnp.full((_L,), jnp.sum(sq), jnp.float32) * inv_k
            var = jnp.maximum(qv - mv * mv, jnp.float32(0.0)) \
                + jnp.float32(1e-20)
            mean_rows.append(mv)
            std_rows.append(var * _rsqrt_nr(var))
        md_v = jnp.full((_L,), jnp.sum(sd), jnp.float32) * inv_k
        mx_v = jnp.full((_L,), jnp.max(maxd), jnp.float32)
        for ci, row in enumerate(mean_rows + std_rows + [md_v, mx_v]):
            out_tile[3 + ci, pl.ds(obase, _L)] = row
            out_tile[3 + ci, pl.ds(obase + _L, _L)] = row

        return carry_g

    lax.fori_loop(0, _SPW, centroid_body, 0)
    ngroups = _SPW // _G
    _drain(_SPW - 2)
    _drain(_SPW - 1)
    _flush_copy((ngroups - 1) % 2, ngroups - 1).start()
    _flush_copy((ngroups - 2) % 2, ngroups - 2).wait()
    _flush_copy((ngroups - 1) % 2, ngroups - 1).wait()


def kernel(xyz, new_xyz, features):
    xyzT = jnp.transpose(xyz, (0, 2, 1)).reshape(_B, 1, 3 * _N)
    newT = jnp.transpose(new_xyz, (0, 2, 1)).reshape(_B * 3, 1, _S)
    featP = jnp.pad(features, ((0, 0), (0, _CP - _C), (0, 0)))
    featT = jnp.transpose(featP, (0, 2, 1)).reshape(_B * _N, _CP)
    mesh = plsc.VectorSubcoreMesh(core_axis_name="c", subcore_axis_name="s")
    f = functools.partial(
        pl.kernel,
        out_type=jax.ShapeDtypeStruct((_B, _NCH, _S * _K), jnp.float32),
        mesh=mesh,
        compiler_params=pltpu.CompilerParams(needs_layout_passes=False, use_tc_tiling_on_sc=False),
        scratch_types=[
            pltpu.VMEM((3 * _N,), jnp.float32),
            pltpu.VMEM((3 * _SPW,), jnp.float32),
            pltpu.VMEM((320,), jnp.int32),
            pltpu.VMEM((2, _K), jnp.int32),
            pltpu.VMEM((2, _K, _CP), jnp.float32),
            pltpu.VMEM((2, _NCH, _G * _K), jnp.float32),
            pltpu.SemaphoreType.DMA,
            pltpu.SemaphoreType.DMA,
            pltpu.SemaphoreType.DMA,
        ],
    )(_sc_body)
    return f(xyzT, newT, featT).reshape(_B, _NCH, _S, _K)
